# baseline (speedup 1.0000x reference)
import functools

import jax
import jax.numpy as jnp
from jax import lax
from jax.experimental import pallas as pl
from jax.experimental.pallas import tpu as pltpu
from jax.experimental.pallas import tpu_sc as plsc

_B, _T, _P, _F = 16, 50, 200, 128
_N = _B * _T
_G = 100
_NVEC = _N // 16


def _sc_gather_body(tl_hbm, tt_hbm, te_hbm, tlv, ttv, tev):
    cid = lax.axis_index("c")
    sid = lax.axis_index("s")
    wid = sid * 2 + cid

    pltpu.sync_copy(tl_hbm, tlv)
    pltpu.sync_copy(tt_hbm, ttv)

    lane = lax.iota(jnp.int32, 16)
    zeros = jnp.zeros((16,), jnp.int32)
    for k in range(2):
        vidx = wid * 2 + k

        @pl.when(vidx < _NVEC)
        def _():
            s = lane + vidx * 16
            tl16 = plsc.load_gather(tlv, [s // _T, s % _T])
            tev[0, :] = plsc.load_gather(ttv, [zeros, tl16 // 3])
            pltpu.sync_copy(tev, te_hbm.at[pl.ds(vidx, 1)])


_sc_gather = functools.partial(
    pl.kernel,
    mesh=plsc.VectorSubcoreMesh(core_axis_name="c", subcore_axis_name="s"),
    compiler_params=pltpu.CompilerParams(needs_layout_passes=False),
    out_type=jax.ShapeDtypeStruct((_NVEC, 16), jnp.float32),
    scratch_types=[
        pltpu.VMEM((_B, _T), jnp.int32),
        pltpu.VMEM((1, 8), jnp.float32),
        pltpu.VMEM((1, 16), jnp.float32),
    ],
)(_sc_gather_body)


def _tc_body(te_sp, x_ref, pt_ref, ft_ref, o_ref):
    g = pl.program_id(0)
    pf = pt_ref[...] + ft_ref[...]
    for r in range(_G):
        o_ref[r] = x_ref[r] + (pf + te_sp[g * _G + r])


@jax.jit
def kernel(x, time_list, time_table, point_table, f_table):
    x3 = x.reshape(_N, _P, _F)
    tl = time_list.astype(jnp.int32)
    pt = point_table.reshape(_P, 1)
    ft = f_table.reshape(1, _F)

    te = _sc_gather(tl, time_table.reshape(1, 8)).reshape(_N)

    grid_spec = pltpu.PrefetchScalarGridSpec(
        num_scalar_prefetch=1,
        grid=(_N // _G,),
        in_specs=[
            pl.BlockSpec((_G, _P, _F), lambda g, te_sp: (g, 0, 0)),
            pl.BlockSpec((_P, 1), lambda g, te_sp: (0, 0)),
            pl.BlockSpec((1, _F), lambda g, te_sp: (0, 0)),
        ],
        out_specs=pl.BlockSpec((_G, _P, _F), lambda g, te_sp: (g, 0, 0)),
    )
    out = pl.pallas_call(
        _tc_body,
        grid_spec=grid_spec,
        out_shape=jax.ShapeDtypeStruct((_N, _P, _F), jnp.float32),
    )(te, x3, pt, ft)
    return out.reshape(_B, _T, _P, _F)


# speedup vs baseline: 1.0126x; 1.0126x over previous
import functools

import jax
import jax.numpy as jnp
from jax import lax
from jax.experimental import pallas as pl
from jax.experimental.pallas import tpu as pltpu
from jax.experimental.pallas import tpu_sc as plsc

_B, _T, _P, _F = 16, 50, 200, 128
_N = _B * _T
_G = 100
_NVEC = _N // 16


def _sc_gather_body(tl_hbm, tt_hbm, te_hbm, tlv, ttv, tev, sem0, sem1):
    cid = lax.axis_index("c")
    sid = lax.axis_index("s")
    wid = sid * 2 + cid

    @pl.when(wid < _NVEC // 2)
    def _():
        ld0 = pltpu.async_copy(tl_hbm, tlv, sem0)
        ld1 = pltpu.async_copy(tt_hbm, ttv, sem1)
        ld0.wait()
        ld1.wait()

        lane = lax.iota(jnp.int32, 16)
        zeros = jnp.zeros((16,), jnp.int32)
        for k in range(2):
            s = lane + (wid * 2 + k) * 16
            tl16 = plsc.load_gather(tlv, [s // _T, s % _T])
            tev[k, :] = plsc.load_gather(ttv, [zeros, tl16 // 3])
        pltpu.sync_copy(tev, te_hbm.at[pl.ds(wid * 2, 2)])


_sc_gather = functools.partial(
    pl.kernel,
    mesh=plsc.VectorSubcoreMesh(core_axis_name="c", subcore_axis_name="s"),
    compiler_params=pltpu.CompilerParams(needs_layout_passes=False),
    out_type=jax.ShapeDtypeStruct((_NVEC, 16), jnp.float32),
    scratch_types=[
        pltpu.VMEM((_B, _T), jnp.int32),
        pltpu.VMEM((1, 8), jnp.float32),
        pltpu.VMEM((2, 16), jnp.float32),
        pltpu.SemaphoreType.DMA,
        pltpu.SemaphoreType.DMA,
    ],
)(_sc_gather_body)


def _tc_body(te_sp, x_ref, pt_ref, ft_ref, o_ref):
    g = pl.program_id(0)
    pf = pt_ref[...] + ft_ref[...]
    for r in range(_G):
        o_ref[r] = x_ref[r] + (pf + te_sp[g * _G + r])


@jax.jit
def kernel(x, time_list, time_table, point_table, f_table):
    x3 = x.reshape(_N, _P, _F)
    tl = time_list.astype(jnp.int32)
    pt = point_table.reshape(_P, 1)
    ft = f_table.reshape(1, _F)

    te = _sc_gather(tl, time_table.reshape(1, 8)).reshape(_N)

    grid_spec = pltpu.PrefetchScalarGridSpec(
        num_scalar_prefetch=1,
        grid=(_N // _G,),
        in_specs=[
            pl.BlockSpec((_G, _P, _F), lambda g, te_sp: (g, 0, 0)),
            pl.BlockSpec((_P, 1), lambda g, te_sp: (0, 0)),
            pl.BlockSpec((1, _F), lambda g, te_sp: (0, 0)),
        ],
        out_specs=pl.BlockSpec((_G, _P, _F), lambda g, te_sp: (g, 0, 0)),
    )
    out = pl.pallas_call(
        _tc_body,
        grid_spec=grid_spec,
        out_shape=jax.ShapeDtypeStruct((_N, _P, _F), jnp.float32),
    )(te, x3, pt, ft)
    return out.reshape(_B, _T, _P, _F)
